# baseline (speedup 1.0000x reference)
import functools

import jax
import jax.numpy as jnp
from jax import lax
from jax.experimental import pallas as pl
from jax.experimental.pallas import tpu as pltpu
from jax.experimental.pallas import tpu_sc as plsc

NC = 2
NS = 16
NW = NC * NS

GW = 128
R = 4


def _emb_body(rows_per_w, n_chunks, D,
              table_hbm, idx_hbm, out_hbm, idx_v, rows_v, gsem):
    wid = lax.axis_index("s") * NC + lax.axis_index("c")
    base = wid * rows_per_w

    @pl.loop(0, n_chunks)
    def _chunk(c):
        row0 = base + c * R
        pltpu.sync_copy(idx_hbm.at[pl.ds(row0, R)], idx_v)
        copies = []
        for j in range(R):
            copies.append(
                pltpu.async_copy(table_hbm.at[idx_v.at[j]], rows_v.at[j], gsem)
            )
        for cp in copies:
            cp.wait()
        pltpu.sync_copy(rows_v, out_hbm.at[pl.ds(row0, R)])


def kernel(x, embedding):
    B0, B1 = x.shape
    V, D = embedding.shape
    B = B0 * B1
    rows_total = B // GW
    rows_per_w = rows_total // NW
    n_chunks = rows_per_w // R

    idx = x.reshape(rows_total, GW).astype(jnp.int32)

    mesh = plsc.VectorSubcoreMesh(core_axis_name="c", subcore_axis_name="s")
    body = functools.partial(_emb_body, rows_per_w, n_chunks, D)
    out = pl.kernel(
        body,
        out_type=jax.ShapeDtypeStruct((rows_total, GW, D), jnp.float32),
        mesh=mesh,
        scratch_types=[
            pltpu.VMEM((R, GW), jnp.int32),
            pltpu.VMEM((R, GW, D), jnp.float32),
            pltpu.SemaphoreType.DMA,
        ],
        compiler_params=pltpu.CompilerParams(use_tc_tiling_on_sc=False),
    )(embedding, idx)
    return out.reshape(B0, B1, D)


# speedup vs baseline: 1.0468x; 1.0468x over previous
import functools

import jax
import jax.numpy as jnp
from jax import lax
from jax.experimental import pallas as pl
from jax.experimental.pallas import tpu as pltpu
from jax.experimental.pallas import tpu_sc as plsc

NC = 2
NS = 16
NW = NC * NS

GW = 128
R = 5


def _emb_body(rows_per_w, n_chunks, D,
              table_hbm, idx_hbm, out_hbm, idx_v, rows_v, gsem, osem):
    wid = lax.axis_index("s") * NC + lax.axis_index("c")
    base = wid * rows_per_w

    pltpu.sync_copy(idx_hbm.at[pl.ds(base, rows_per_w)], idx_v)

    def fire_gathers(buf, c):
        for j in range(R):
            pltpu.async_copy(
                table_hbm.at[idx_v.at[c * R + j]], rows_v.at[buf].at[j], gsem
            )

    def drain_gathers(buf, c):
        for j in range(R):
            pltpu.make_async_copy(
                table_hbm.at[idx_v.at[c * R + j]], rows_v.at[buf].at[j], gsem
            ).wait()

    def out_slice(c):
        return out_hbm.at[pl.ds(base + c * R, R)]

    def fire_write(buf, c):
        pltpu.async_copy(rows_v.at[buf], out_slice(c), osem)

    def wait_write(buf, c):
        pltpu.make_async_copy(rows_v.at[buf], out_slice(c), osem).wait()

    fire_gathers(0, 0)

    @pl.loop(0, n_chunks, step=2)
    def _super(s):
        for b in range(2):
            c = s + b
            nxt = (b + 1) % 2
            if b == 0:
                @pl.when(s > 0)
                def _():
                    wait_write(nxt, c - 1)
            else:
                wait_write(nxt, c - 1)
            if b == 0:
                fire_gathers(nxt, c + 1)
            else:
                @pl.when(s < n_chunks - 2)
                def _():
                    fire_gathers(nxt, c + 1)
            drain_gathers(b, c)
            fire_write(b, c)

    wait_write((n_chunks - 1) % 2, n_chunks - 1)


def kernel(x, embedding):
    B0, B1 = x.shape
    V, D = embedding.shape
    B = B0 * B1
    rows_total = B // GW
    rows_per_w = rows_total // NW
    n_chunks = rows_per_w // R

    idx = x.reshape(rows_total, GW).astype(jnp.int32)

    mesh = plsc.VectorSubcoreMesh(core_axis_name="c", subcore_axis_name="s")
    body = functools.partial(_emb_body, rows_per_w, n_chunks, D)
    out = pl.kernel(
        body,
        out_type=jax.ShapeDtypeStruct((rows_total, GW, D), jnp.float32),
        mesh=mesh,
        scratch_types=[
            pltpu.VMEM((rows_per_w, GW), jnp.int32),
            pltpu.VMEM((2, R, GW, D), jnp.float32),
            pltpu.SemaphoreType.DMA,
            pltpu.SemaphoreType.DMA,
        ],
        compiler_params=pltpu.CompilerParams(use_tc_tiling_on_sc=False),
    )(embedding, idx)
    return out.reshape(B0, B1, D)


# speedup vs baseline: 1.2305x; 1.1755x over previous
import functools
import jax
import jax.numpy as jnp
from jax import lax
from jax.experimental import pallas as pl
from jax.experimental.pallas import tpu as pltpu
from jax.experimental.pallas import tpu_sc as plsc

NC, NS = 2, 16
NW = NC * NS
GW = 128
R = 4


def _body(rows_per_w, n_chunks, D,
          table_hbm, idx_hbm, out_hbm, idx_v, pad_v, gsem):
    wid = lax.axis_index("s") * NC + lax.axis_index("c")
    base = wid * rows_per_w

    @pl.loop(0, n_chunks)
    def _chunk(c):
        row0 = base + c * R
        pltpu.sync_copy(idx_hbm.at[pl.ds(row0, R)], idx_v)
        copies = []
        for j in range(R):
            copies.append(
                pltpu.async_copy(table_hbm.at[idx_v.at[j]], pad_v.at[j], gsem))
        for cp in copies:
            cp.wait()
        pltpu.sync_copy(pad_v, out_hbm.at[pl.ds(row0, R)])


def kernel(x, embedding):
    B0, B1 = x.shape
    V, D = embedding.shape
    B = B0 * B1
    rows_total = B // GW
    rows_per_w = rows_total // NW
    n_chunks = rows_per_w // R

    idx = x.reshape(rows_total, GW).astype(jnp.int32)
    tablew = jnp.pad(embedding, ((0, 0), (0, 128 - D)))

    mesh = plsc.VectorSubcoreMesh(core_axis_name="c", subcore_axis_name="s")
    body = functools.partial(_body, rows_per_w, n_chunks, D)
    out = pl.kernel(
        body,
        out_type=jax.ShapeDtypeStruct((rows_total, GW, 128), jnp.float32),
        mesh=mesh,
        scratch_types=[
            pltpu.VMEM((R, GW), jnp.int32),
            pltpu.VMEM((R, GW, 128), jnp.float32),
            pltpu.SemaphoreType.DMA,
        ],
        compiler_params=pltpu.CompilerParams(use_tc_tiling_on_sc=True),
    )(tablew, idx)
    return out[:, :, :D].reshape(B0, B1, D)


# speedup vs baseline: 1.2777x; 1.0383x over previous
import functools

import jax
import jax.numpy as jnp
from jax import lax
from jax.experimental import pallas as pl
from jax.experimental.pallas import tpu as pltpu
from jax.experimental.pallas import tpu_sc as plsc

NC = 2
NS = 16
NW = NC * NS

GW = 128
R = 2


def _body(rows_per_w, n_chunks, D,
          table_hbm, idx_hbm, out_hbm, idx_v, pad_v, gsem, osem):
    wid = lax.axis_index("s") * NC + lax.axis_index("c")
    base = wid * rows_per_w

    pltpu.sync_copy(idx_hbm.at[pl.ds(base, rows_per_w)], idx_v)

    def fire_gathers(buf, c):
        for j in range(R):
            pltpu.async_copy(
                table_hbm.at[idx_v.at[c * R + j]], pad_v.at[buf].at[j], gsem)

    def drain_gathers(buf, c):
        for j in range(R):
            pltpu.make_async_copy(
                table_hbm.at[idx_v.at[c * R + j]], pad_v.at[buf].at[j], gsem
            ).wait()

    def out_slice(c):
        return out_hbm.at[pl.ds(base + c * R, R)]

    def fire_write(buf, c):
        pltpu.async_copy(pad_v.at[buf], out_slice(c), osem)

    def wait_write(buf, c):
        pltpu.make_async_copy(pad_v.at[buf], out_slice(c), osem).wait()

    fire_gathers(0, 0)

    @pl.loop(0, n_chunks, step=2)
    def _super(s):
        for b in range(2):
            c = s + b
            nxt = (b + 1) % 2
            if b == 0:
                @pl.when(s > 0)
                def _():
                    wait_write(nxt, c - 1)
            else:
                wait_write(nxt, c - 1)
            if b == 0:
                fire_gathers(nxt, c + 1)
            else:
                @pl.when(s < n_chunks - 2)
                def _():
                    fire_gathers(nxt, c + 1)
            drain_gathers(b, c)
            fire_write(b, c)

    wait_write((n_chunks - 1) % 2, n_chunks - 1)


def kernel(x, embedding):
    B0, B1 = x.shape
    V, D = embedding.shape
    B = B0 * B1
    rows_total = B // GW
    rows_per_w = rows_total // NW
    n_chunks = rows_per_w // R

    idx = x.reshape(rows_total, GW).astype(jnp.int32)
    tablew = jnp.pad(embedding, ((0, 0), (0, 128 - D)))

    mesh = plsc.VectorSubcoreMesh(core_axis_name="c", subcore_axis_name="s")
    body = functools.partial(_body, rows_per_w, n_chunks, D)
    out = pl.kernel(
        body,
        out_type=jax.ShapeDtypeStruct((rows_total, GW, 128), jnp.float32),
        mesh=mesh,
        scratch_types=[
            pltpu.VMEM((rows_per_w, GW), jnp.int32),
            pltpu.VMEM((2, R, GW, 128), jnp.float32),
            pltpu.SemaphoreType.DMA,
            pltpu.SemaphoreType.DMA,
        ],
        compiler_params=pltpu.CompilerParams(use_tc_tiling_on_sc=True),
    )(tablew, idx)
    return out[:, :, :D].reshape(B0, B1, D)


# speedup vs baseline: 1.2790x; 1.0010x over previous
import functools

import jax
import jax.numpy as jnp
from jax import lax
from jax.experimental import pallas as pl
from jax.experimental.pallas import tpu as pltpu
from jax.experimental.pallas import tpu_sc as plsc

NC = 2
NS = 16
NW = NC * NS

GW = 128
NBUF = 5
KAHEAD = 3


def _body(rows_per_w, n_chunks, D,
          table_hbm, idx_hbm, out_hbm, idx_v, pad_v, gsem, osem):
    wid = lax.axis_index("s") * NC + lax.axis_index("c")
    base = wid * rows_per_w

    pltpu.sync_copy(idx_hbm.at[pl.ds(base, rows_per_w)], idx_v)

    def fire_gather(buf, c):
        pltpu.async_copy(table_hbm.at[idx_v.at[c]], pad_v.at[buf], gsem)

    def drain_gather(buf, c):
        pltpu.make_async_copy(
            table_hbm.at[idx_v.at[c]], pad_v.at[buf], gsem).wait()

    def fire_write(buf, c):
        pltpu.async_copy(pad_v.at[buf], out_hbm.at[base + c], osem)

    def wait_write(buf, c):
        pltpu.make_async_copy(
            pad_v.at[buf], out_hbm.at[base + c], osem).wait()

    for p in range(KAHEAD):
        fire_gather(p, p)

    @pl.loop(0, n_chunks, step=NBUF)
    def _super(s):
        for b in range(NBUF):
            c = s + b
            fbuf = (b + KAHEAD) % NBUF
            if b < NBUF - KAHEAD:
                @pl.when(s > 0)
                def _():
                    wait_write(fbuf, c - (NBUF - KAHEAD))
            else:
                wait_write(fbuf, c - (NBUF - KAHEAD))
            if b < NBUF - KAHEAD:
                fire_gather(fbuf, c + KAHEAD)
            else:
                @pl.when(s < n_chunks - NBUF)
                def _():
                    fire_gather(fbuf, c + KAHEAD)
            drain_gather(b, c)
            fire_write(b, c)

    wait_write((n_chunks - 2) % NBUF, n_chunks - 2)
    wait_write((n_chunks - 1) % NBUF, n_chunks - 1)


def kernel(x, embedding):
    B0, B1 = x.shape
    V, D = embedding.shape
    B = B0 * B1
    rows_total = B // GW
    rows_per_w = rows_total // NW
    n_chunks = rows_per_w

    idx = x.reshape(rows_total, GW).astype(jnp.int32)
    tablew = jnp.pad(embedding, ((0, 0), (0, 128 - D)))

    mesh = plsc.VectorSubcoreMesh(core_axis_name="c", subcore_axis_name="s")
    body = functools.partial(_body, rows_per_w, n_chunks, D)
    out = pl.kernel(
        body,
        out_type=jax.ShapeDtypeStruct((rows_total, GW, 128), jnp.float32),
        mesh=mesh,
        scratch_types=[
            pltpu.VMEM((rows_per_w, GW), jnp.int32),
            pltpu.VMEM((NBUF, GW, 128), jnp.float32),
            pltpu.SemaphoreType.DMA,
            pltpu.SemaphoreType.DMA,
        ],
        compiler_params=pltpu.CompilerParams(use_tc_tiling_on_sc=True),
    )(tablew, idx)
    return out[:, :, :D].reshape(B0, B1, D)


# speedup vs baseline: 1.4803x; 1.1574x over previous
import functools
import jax
import jax.numpy as jnp
from jax import lax
from jax.experimental import pallas as pl
from jax.experimental.pallas import tpu as pltpu
from jax.experimental.pallas import tpu_sc as plsc

NC, NS = 2, 16
NW = NC * NS
GW = 128
R = 4


def _body(rows_per_w, n_chunks, D,
          table_hbm, idx_hbm, out_hbm, idx_v, rows_v, gsem, osem):
    wid = lax.axis_index("s") * NC + lax.axis_index("c")
    base = wid * rows_per_w

    @pl.loop(0, n_chunks)
    def _chunk(c):
        row0 = base + c * R
        pltpu.sync_copy(idx_hbm.at[pl.ds(row0, R)], idx_v)

        @pl.loop(0, R * GW // 16)
        def _grp(g):
            j = g // (GW // 16)
            g0 = g % (GW // 16)
            v = idx_v[j, pl.ds(g0 * 16, 16)]
            for l in range(16):
                pltpu.async_copy(
                    table_hbm.at[pl.ds(v[l], 1)],
                    rows_v.at[j].at[pl.ds(g0 * 16 + l, 1)], gsem)

        pltpu.make_async_copy(
            table_hbm.at[pl.ds(0, R * GW)],
            rows_v.reshape(R * GW, D), gsem).wait()
        pltpu.sync_copy(rows_v, out_hbm.at[pl.ds(row0, R)])


def kernel(x, embedding):
    B0, B1 = x.shape
    V, D = embedding.shape
    B = B0 * B1
    rows_total = B // GW
    rows_per_w = rows_total // NW
    n_chunks = rows_per_w // R

    idx = x.reshape(rows_total, GW).astype(jnp.int32)

    mesh = plsc.VectorSubcoreMesh(core_axis_name="c", subcore_axis_name="s")
    body = functools.partial(_body, rows_per_w, n_chunks, D)
    out = pl.kernel(
        body,
        out_type=jax.ShapeDtypeStruct((rows_total, GW, D), jnp.float32),
        mesh=mesh,
        scratch_types=[
            pltpu.VMEM((R, GW), jnp.int32),
            pltpu.VMEM((R, GW, D), jnp.float32),
            pltpu.SemaphoreType.DMA,
            pltpu.SemaphoreType.DMA,
        ],
        compiler_params=pltpu.CompilerParams(use_tc_tiling_on_sc=True),
    )(embedding, idx)
    return out.reshape(B0, B1, D)


# speedup vs baseline: 1.5663x; 1.0581x over previous
import functools

import jax
import jax.numpy as jnp
from jax import lax
from jax.experimental import pallas as pl
from jax.experimental.pallas import tpu as pltpu
from jax.experimental.pallas import tpu_sc as plsc

NC = 2
NS = 16
NW = NC * NS

GW = 128
NBUF = 5
KAHEAD = 3


def _body(rows_per_w, n_chunks, D,
          table_hbm, idx_hbm, out_hbm, idx_v, rows_v, gsem, osem):
    wid = lax.axis_index("s") * NC + lax.axis_index("c")
    base = wid * rows_per_w

    pltpu.sync_copy(idx_hbm.at[pl.ds(base, rows_per_w)], idx_v)

    def fire_gather(buf, c):
        @pl.loop(0, GW // 16)
        def _grp(g):
            v = idx_v[c, pl.ds(g * 16, 16)]
            for l in range(16):
                pltpu.async_copy(
                    table_hbm.at[pl.ds(v[l], 1)],
                    rows_v.at[buf].at[pl.ds(g * 16 + l, 1)], gsem)

    def drain_gather(buf, c):
        pltpu.make_async_copy(
            table_hbm.at[pl.ds(0, GW)], rows_v.at[buf], gsem).wait()

    def fire_write(buf, c):
        pltpu.async_copy(rows_v.at[buf], out_hbm.at[base + c], osem)

    def wait_write(buf, c):
        pltpu.make_async_copy(
            rows_v.at[buf], out_hbm.at[base + c], osem).wait()

    for p in range(KAHEAD):
        fire_gather(p, p)

    @pl.loop(0, n_chunks, step=NBUF)
    def _super(s):
        for b in range(NBUF):
            c = s + b
            fbuf = (b + KAHEAD) % NBUF
            if b < NBUF - KAHEAD:
                @pl.when(s > 0)
                def _():
                    wait_write(fbuf, c - (NBUF - KAHEAD))
            else:
                wait_write(fbuf, c - (NBUF - KAHEAD))
            if b < NBUF - KAHEAD:
                fire_gather(fbuf, c + KAHEAD)
            else:
                @pl.when(s < n_chunks - NBUF)
                def _():
                    fire_gather(fbuf, c + KAHEAD)
            drain_gather(b, c)
            fire_write(b, c)

    wait_write((n_chunks - 2) % NBUF, n_chunks - 2)
    wait_write((n_chunks - 1) % NBUF, n_chunks - 1)


def kernel(x, embedding):
    B0, B1 = x.shape
    V, D = embedding.shape
    B = B0 * B1
    rows_total = B // GW
    rows_per_w = rows_total // NW
    n_chunks = rows_per_w

    idx = x.reshape(rows_total, GW).astype(jnp.int32)

    mesh = plsc.VectorSubcoreMesh(core_axis_name="c", subcore_axis_name="s")
    body = functools.partial(_body, rows_per_w, n_chunks, D)
    out = pl.kernel(
        body,
        out_type=jax.ShapeDtypeStruct((rows_total, GW, D), jnp.float32),
        mesh=mesh,
        scratch_types=[
            pltpu.VMEM((rows_per_w, GW), jnp.int32),
            pltpu.VMEM((NBUF, GW, D), jnp.float32),
            pltpu.SemaphoreType.DMA,
            pltpu.SemaphoreType.DMA,
        ],
        compiler_params=pltpu.CompilerParams(use_tc_tiling_on_sc=True),
    )(embedding, idx)
    return out.reshape(B0, B1, D)


# speedup vs baseline: 1.8379x; 1.1734x over previous
import functools

import jax
import jax.numpy as jnp
from jax import lax
from jax.experimental import pallas as pl
from jax.experimental.pallas import tpu as pltpu
from jax.experimental.pallas import tpu_sc as plsc

NC = 2
NS = 16
NW = NC * NS

GW = 128
NBUF = 5
KAHEAD = 3


def _body(rows_per_w, n_chunks, D,
          table_hbm, idx_hbm, out_hbm, idx_v, rows_v, gsem, osem):
    wid = lax.axis_index("s") * NC + lax.axis_index("c")
    base = wid * rows_per_w

    pltpu.sync_copy(idx_hbm.at[pl.ds(base, rows_per_w)], idx_v)

    def fire_gather(buf, c):
        @pl.loop(0, GW // 16)
        def _grp(g):
            v = idx_v[c, pl.ds(g * 16, 16)]
            for l in range(16):
                r = v[l]
                pltpu.async_copy(
                    table_hbm.at[pl.ds(r // 8, 1), pl.ds(r % 8, 1)],
                    rows_v.at[buf].at[pl.ds(g * 16 + l, 1)].reshape(1, 1, 64),
                    gsem)

    def drain_gather(buf, c):
        pltpu.make_async_copy(
            table_hbm.at[pl.ds(0, GW // 8)], rows_v.at[buf].reshape(GW // 8, 8, D),
            gsem).wait()

    def fire_write(buf, c):
        pltpu.async_copy(rows_v.at[buf], out_hbm.at[base + c], osem)

    def wait_write(buf, c):
        pltpu.make_async_copy(
            rows_v.at[buf], out_hbm.at[base + c], osem).wait()

    for p in range(KAHEAD):
        fire_gather(p, p)

    @pl.loop(0, n_chunks, step=NBUF)
    def _super(s):
        for b in range(NBUF):
            c = s + b
            fbuf = (b + KAHEAD) % NBUF
            if b < NBUF - KAHEAD:
                @pl.when(s > 0)
                def _():
                    wait_write(fbuf, c - (NBUF - KAHEAD))
            else:
                wait_write(fbuf, c - (NBUF - KAHEAD))
            if b < NBUF - KAHEAD:
                fire_gather(fbuf, c + KAHEAD)
            else:
                @pl.when(s < n_chunks - NBUF)
                def _():
                    fire_gather(fbuf, c + KAHEAD)
            drain_gather(b, c)
            fire_write(b, c)

    wait_write((n_chunks - 2) % NBUF, n_chunks - 2)
    wait_write((n_chunks - 1) % NBUF, n_chunks - 1)


def kernel(x, embedding):
    B0, B1 = x.shape
    V, D = embedding.shape
    B = B0 * B1
    rows_total = B // GW
    rows_per_w = rows_total // NW
    n_chunks = rows_per_w

    idx = x.reshape(rows_total, GW).astype(jnp.int32)

    mesh = plsc.VectorSubcoreMesh(core_axis_name="c", subcore_axis_name="s")
    body = functools.partial(_body, rows_per_w, n_chunks, D)
    out = pl.kernel(
        body,
        out_type=jax.ShapeDtypeStruct((rows_total, GW, D), jnp.float32),
        mesh=mesh,
        scratch_types=[
            pltpu.VMEM((rows_per_w, GW), jnp.int32),
            pltpu.VMEM((NBUF, GW, D), jnp.float32),
            pltpu.SemaphoreType.DMA,
            pltpu.SemaphoreType.DMA,
        ],
        compiler_params=pltpu.CompilerParams(use_tc_tiling_on_sc=True),
    )(embedding.reshape(V // 8, 8, D), idx)
    return out.reshape(B0, B1, D)
